# baseline (speedup 1.0000x reference)
import functools

import jax
import jax.numpy as jnp
from jax import lax
from jax.experimental import pallas as pl
from jax.experimental.pallas import tpu as pltpu
from jax.experimental.pallas import tpu_sc as plsc

N = 100000
E = 3200000
EPS = 1e-06

NC = 2
NS = 16
NW = NC * NS

N_PAD = 102400
NPT = N_PAD // NS
CHUNK = 6400
NCHG = E // CHUNK
NCH = -(-NCHG // NW)

_mesh = plsc.VectorSubcoreMesh(
    core_axis_name="c", subcore_axis_name="s", num_cores=NC, num_subcores=NS
)


def _spmv_body(ei_hbm, a_hbm, resid_hbm, out_hbm,
               ei_v0, ei_v1, srcf0, srcf1, dstf0, dstf1,
               a_v0, a_v1, g_v0, g_v1,
               stage_v, resid_sh, ad_sh, sems):
    ei_v = (ei_v0, ei_v1)
    srcf = (srcf0, srcf1)
    dstf = (dstf0, dstf1)
    a_v = (a_v0, a_v1)
    g_v = (g_v0, g_v1)
    c = lax.axis_index("c")
    s = lax.axis_index("s")
    wid = c * NS + s

    node_base = pl.multiple_of(s * NPT, 8)
    resid_base = pl.multiple_of(jnp.minimum(s * NPT, N - NPT), 8)
    pltpu.sync_copy(resid_hbm.at[pl.ds(resid_base, NPT)], stage_v)
    pltpu.sync_copy(stage_v, resid_sh.at[pl.ds(resid_base, NPT)])

    @plsc.parallel_loop(0, NPT, step=16)
    def _zero(i):
        stage_v[pl.ds(i, 16)] = jnp.zeros((16,), jnp.float32)

    pltpu.sync_copy(stage_v, ad_sh.at[pl.ds(node_base, NPT)])
    plsc.subcore_barrier()

    lin_d = [None] * NCH
    gat_d = [None] * NCH
    sct_d = [None] * NCH

    def chunk_info(ci):
        gci = wid + NW * ci
        if (ci + 1) * NW <= NCHG:
            return pl.multiple_of(gci * CHUNK, 128), None
        valid = gci < NCHG
        base = jnp.where(valid, gci, wid) * CHUNK
        return pl.multiple_of(base, 128), valid

    def lin_start(ci):
        b = ci % 2
        base, _ = chunk_info(ci)
        lin_d[ci] = (
            pltpu.async_copy(ei_hbm.at[:, pl.ds(base, CHUNK)],
                             ei_v[b], sems.at[0, b]),
            pltpu.async_copy(a_hbm.at[pl.ds(base, CHUNK)],
                             a_v[b], sems.at[2, b]),
        )

    lin_start(0)
    for ci in range(NCH):
        b = ci % 2
        for dsc in lin_d[ci]:
            dsc.wait()

        et, sf, df = ei_v[b], srcf[b], dstf[b]

        @plsc.parallel_loop(0, CHUNK, step=16, unroll=8)
        def _deint(i):
            sl = pl.ds(i, 16)
            sf[sl] = et[0, sl]

        gat_d[ci] = pltpu.async_copy(resid_sh.at[sf],
                                     g_v[b], sems.at[3, b])
        if ci >= 1 and sct_d[ci - 1] is not None:
            sct_d[ci - 1].wait()
            sct_d[ci - 1] = None
        if ci + 1 < NCH:
            lin_start(ci + 1)
        gat_d[ci].wait()

        gb, ab = g_v[b], a_v[b]
        _, valid = chunk_info(ci)

        if valid is None:
            @plsc.parallel_loop(0, CHUNK, step=16, unroll=8)
            def _mul(i):
                sl = pl.ds(i, 16)
                gb[sl] = gb[sl] * ab[sl]
                df[sl] = et[1, sl]
        else:
            scale = jnp.where(valid, 1.0, 0.0).astype(jnp.float32)

            @plsc.parallel_loop(0, CHUNK, step=16, unroll=8)
            def _mul(i):
                sl = pl.ds(i, 16)
                gb[sl] = gb[sl] * ab[sl] * scale
                df[sl] = et[1, sl]

        sct_d[ci] = pltpu.async_copy(gb, ad_sh.at[df],
                                     sems.at[4, b], add=True)
    for dct in sct_d:
        if dct is not None:
            dct.wait()
    plsc.subcore_barrier()

    pltpu.sync_copy(ad_sh.at[pl.ds(node_base, NPT)], stage_v)
    pltpu.sync_copy(stage_v, out_hbm.at[c, pl.ds(node_base, NPT)])


_spmv_sc = functools.partial(
    pl.kernel,
    out_type=jax.ShapeDtypeStruct((NC, N_PAD), jnp.float32),
    mesh=_mesh,
    scratch_types=[
        pltpu.VMEM((2, CHUNK), jnp.int32),
        pltpu.VMEM((2, CHUNK), jnp.int32),
        pltpu.VMEM((CHUNK,), jnp.int32),
        pltpu.VMEM((CHUNK,), jnp.int32),
        pltpu.VMEM((CHUNK,), jnp.int32),
        pltpu.VMEM((CHUNK,), jnp.int32),
        pltpu.VMEM((CHUNK,), jnp.float32),
        pltpu.VMEM((CHUNK,), jnp.float32),
        pltpu.VMEM((CHUNK,), jnp.float32),
        pltpu.VMEM((CHUNK,), jnp.float32),
        pltpu.VMEM((NPT,), jnp.float32),
        pltpu.VMEM_SHARED((N_PAD,), jnp.float32),
        pltpu.VMEM_SHARED((N_PAD,), jnp.float32),
        pltpu.SemaphoreType.DMA((5, 2)),
    ],
)(_spmv_body)


def _loss_body(p_ref, d_ref, out_ref):
    ad = p_ref[0, :] + p_ref[1, :]
    e = d_ref[...] - ad
    err = jnp.sum(e * e)
    gt = jnp.sum(ad * ad)
    out_ref[...] = jnp.broadcast_to(err / (gt + EPS), (1, 1))


_loss_tc = pl.pallas_call(
    _loss_body,
    out_shape=jax.ShapeDtypeStruct((1, 1), jnp.float32),
)


def kernel(residual, edge_index, matrix_values, d, L_values):
    ei = edge_index.astype(jnp.int32)
    dpad = jnp.pad(d[:, 0], (0, N_PAD - N))
    partial = _spmv_sc(ei, matrix_values.astype(jnp.float32), residual[:, 0])
    return partial[0, 0] + dpad[0]


# speedup vs baseline: 1.0136x; 1.0136x over previous
import functools

import jax
import jax.numpy as jnp
from jax import lax
from jax.experimental import pallas as pl
from jax.experimental.pallas import tpu as pltpu
from jax.experimental.pallas import tpu_sc as plsc

N = 100000
E = 3200000
EPS = 1e-06

NC = 2
NS = 16
NW = NC * NS

N_PAD = 102400
NPT = N_PAD // NS
CHUNK = 6400
NCHG = E // CHUNK
NCH = -(-NCHG // NW)

_mesh = plsc.VectorSubcoreMesh(
    core_axis_name="c", subcore_axis_name="s", num_cores=NC, num_subcores=NS
)


def _spmv_body(ei_hbm, a_hbm, resid_hbm, out_hbm,
               ei_v0, ei_v1, srcf0, srcf1, dstf0, dstf1,
               a_v0, a_v1, g_v0, g_v1,
               stage_v, resid_sh, ad_sh, sems):
    ei_v = (ei_v0, ei_v1)
    srcf = (srcf0, srcf1)
    dstf = (dstf0, dstf1)
    a_v = (a_v0, a_v1)
    g_v = (g_v0, g_v1)
    c = lax.axis_index("c")
    s = lax.axis_index("s")
    wid = c * NS + s

    lin_d = [None] * NCH
    gat_d = [None] * NCH
    sct_d = [None] * NCH

    def chunk_info(ci):
        gci = wid + NW * ci
        if (ci + 1) * NW <= NCHG:
            return pl.multiple_of(gci * CHUNK, 128), None
        valid = gci < NCHG
        base = jnp.where(valid, gci, wid) * CHUNK
        return pl.multiple_of(base, 128), valid

    def lin_start(ci):
        b = ci % 2
        base, _ = chunk_info(ci)
        lin_d[ci] = (
            pltpu.async_copy(ei_hbm.at[:, pl.ds(base, CHUNK)],
                             ei_v[b], sems.at[0, b]),
            pltpu.async_copy(a_hbm.at[pl.ds(base, CHUNK)],
                             a_v[b], sems.at[2, b]),
        )

    lin_start(0)
    lin_start(1)

    node_base = pl.multiple_of(s * NPT, 8)
    pltpu.sync_copy(resid_hbm.at[pl.ds(node_base, NPT)], stage_v)
    pltpu.sync_copy(stage_v, resid_sh.at[pl.ds(node_base, NPT)])

    @plsc.parallel_loop(0, NPT, step=16)
    def _zero(i):
        stage_v[pl.ds(i, 16)] = jnp.zeros((16,), jnp.float32)

    pltpu.sync_copy(stage_v, ad_sh.at[pl.ds(node_base, NPT)])
    plsc.subcore_barrier()

    for ci in range(NCH):
        b = ci % 2
        for dsc in lin_d[ci]:
            dsc.wait()

        et, sf, df = ei_v[b], srcf[b], dstf[b]

        @plsc.parallel_loop(0, CHUNK, step=16, unroll=8)
        def _deint(i):
            sl = pl.ds(i, 16)
            sf[sl] = et[0, sl]
            df[sl] = et[1, sl]

        gat_d[ci] = pltpu.async_copy(resid_sh.at[sf],
                                     g_v[b], sems.at[3, b])
        if ci >= 1 and sct_d[ci - 1] is not None:
            sct_d[ci - 1].wait()
            sct_d[ci - 1] = None
        if ci + 1 < NCH and lin_d[ci + 1] is None:
            lin_start(ci + 1)
        gat_d[ci].wait()

        gb, ab = g_v[b], a_v[b]
        _, valid = chunk_info(ci)

        if valid is None:
            @plsc.parallel_loop(0, CHUNK, step=16, unroll=8)
            def _mul(i):
                sl = pl.ds(i, 16)
                gb[sl] = gb[sl] * ab[sl]
        else:
            scale = jnp.where(valid, 1.0, 0.0).astype(jnp.float32)

            @plsc.parallel_loop(0, CHUNK, step=16, unroll=8)
            def _mul(i):
                sl = pl.ds(i, 16)
                gb[sl] = gb[sl] * ab[sl] * scale

        sct_d[ci] = pltpu.async_copy(gb, ad_sh.at[df],
                                     sems.at[4, b], add=True)
    for dct in sct_d:
        if dct is not None:
            dct.wait()
    plsc.subcore_barrier()

    pltpu.sync_copy(ad_sh.at[pl.ds(node_base, NPT)], stage_v)
    pltpu.sync_copy(stage_v, out_hbm.at[c, pl.ds(node_base, NPT)])


_spmv_sc = functools.partial(
    pl.kernel,
    out_type=jax.ShapeDtypeStruct((NC, N_PAD), jnp.float32),
    mesh=_mesh,
    scratch_types=[
        pltpu.VMEM((2, CHUNK), jnp.int32),
        pltpu.VMEM((2, CHUNK), jnp.int32),
        pltpu.VMEM((CHUNK,), jnp.int32),
        pltpu.VMEM((CHUNK,), jnp.int32),
        pltpu.VMEM((CHUNK,), jnp.int32),
        pltpu.VMEM((CHUNK,), jnp.int32),
        pltpu.VMEM((CHUNK,), jnp.float32),
        pltpu.VMEM((CHUNK,), jnp.float32),
        pltpu.VMEM((CHUNK,), jnp.float32),
        pltpu.VMEM((CHUNK,), jnp.float32),
        pltpu.VMEM((NPT,), jnp.float32),
        pltpu.VMEM_SHARED((N_PAD,), jnp.float32),
        pltpu.VMEM_SHARED((N_PAD,), jnp.float32),
        pltpu.SemaphoreType.DMA((5, 2)),
    ],
)(_spmv_body)


def _loss_body(p_ref, d_ref, out_ref):
    ad = p_ref[0] + p_ref[1]
    e = d_ref[...] - ad
    err = jnp.sum(e * e)
    gt = jnp.sum(ad * ad)
    out_ref[...] = jnp.broadcast_to(err / (gt + EPS), (1, 1))


_loss_tc = pl.pallas_call(
    _loss_body,
    out_shape=jax.ShapeDtypeStruct((1, 1), jnp.float32),
)


def kernel(residual, edge_index, matrix_values, d, L_values):
    ei = edge_index.astype(jnp.int32)
    resid = jnp.pad(residual[:, 0], (0, N_PAD - N))
    dpad = jnp.pad(d[:, 0], (0, N_PAD - N)).reshape(N_PAD // 128, 128)
    partial = _spmv_sc(ei, matrix_values.astype(jnp.float32), resid)
    p3 = partial.reshape(NC, N_PAD // 128, 128)
    return _loss_tc(p3, dpad)[0, 0]
